# baseline (speedup 1.0000x reference)
import functools

import jax
import jax.numpy as jnp
from jax.experimental import pallas as pl
from jax.experimental.pallas import tpu as pltpu

_EPS = 1e-12


def _centroid_kernel(x_ref, c_ref, o_ref, cn_ref, cb_ref):
    p, d = c_ref.shape

    @pl.when(pl.program_id(0) == 0)
    def _():
        c = c_ref[...]
        cn = c * jax.lax.rsqrt(
            jnp.maximum(jnp.sum(c * c, axis=1, keepdims=True), _EPS * _EPS)
        )
        cn_ref[...] = cn.astype(jnp.bfloat16)
        lane = jax.lax.broadcasted_iota(jnp.int32, (p, 128), 1)
        ones_col = jnp.where(lane == 0, 1.0, 0.0).astype(jnp.bfloat16)
        cb_ref[...] = jnp.concatenate([c.astype(jnp.bfloat16), ones_col], axis=1)

    xb = x_ref[...]
    xn = xb * jax.lax.rsqrt(
        jnp.maximum(jnp.sum(xb * xb, axis=1, keepdims=True), _EPS * _EPS)
    )

    sims = jax.lax.dot_general(
        xn.astype(jnp.bfloat16),
        cn_ref[...],
        (((1,), (1,)), ((), ())),
        preferred_element_type=jnp.float32,
    )
    e = jnp.exp(sims.astype(jnp.bfloat16))
    ctx = jnp.dot(e, cb_ref[...], preferred_element_type=jnp.float32)
    o_ref[...] = ctx[:, :d] / ctx[:, d : d + 1]


@functools.partial(jax.jit, static_argnames=("block_n",))
def _centroid_layer(x, centroid_emb, block_n=512):
    n, d = x.shape
    p, _ = centroid_emb.shape
    return pl.pallas_call(
        _centroid_kernel,
        grid=(n // block_n,),
        in_specs=[
            pl.BlockSpec((block_n, d), lambda i: (i, 0)),
            pl.BlockSpec((p, d), lambda i: (0, 0)),
        ],
        out_specs=pl.BlockSpec((block_n, d), lambda i: (i, 0)),
        out_shape=jax.ShapeDtypeStruct((n, d), jnp.float32),
        scratch_shapes=[
            pltpu.VMEM((p, d), jnp.bfloat16),
            pltpu.VMEM((p, d + 128), jnp.bfloat16),
        ],
    )(x, centroid_emb)


def kernel(x, centroid_emb):
    return _centroid_layer(x, centroid_emb)


# speedup vs baseline: 1.1388x; 1.1388x over previous
import functools

import jax
import jax.numpy as jnp
from jax.experimental import pallas as pl
from jax.experimental.pallas import tpu as pltpu

_EPS = 1e-12


def _centroid_kernel(x_ref, c_ref, o_ref, cn_ref, cb_ref):
    p, d = c_ref.shape

    @pl.when(pl.program_id(0) == 0)
    def _():
        c = c_ref[...]
        cn = c * jax.lax.rsqrt(
            jnp.maximum(jnp.sum(c * c, axis=1, keepdims=True), _EPS * _EPS)
        )
        cn_ref[...] = cn.astype(jnp.bfloat16)
        cb_ref[...] = c.astype(jnp.bfloat16)

    xb = x_ref[...]
    xn = xb * jax.lax.rsqrt(
        jnp.maximum(jnp.sum(xb * xb, axis=1, keepdims=True), _EPS * _EPS)
    )

    sims = jax.lax.dot_general(
        xn.astype(jnp.bfloat16),
        cn_ref[...],
        (((1,), (1,)), ((), ())),
        preferred_element_type=jnp.float32,
    )
    e = jnp.exp(sims)
    s = jnp.sum(e, axis=1, keepdims=True)
    ctx = jnp.dot(
        e.astype(jnp.bfloat16), cb_ref[...], preferred_element_type=jnp.float32
    )
    o_ref[...] = ctx / s


@functools.partial(jax.jit, static_argnames=("block_n",))
def _centroid_layer(x, centroid_emb, block_n=512):
    n, d = x.shape
    p, _ = centroid_emb.shape
    return pl.pallas_call(
        _centroid_kernel,
        grid=(n // block_n,),
        in_specs=[
            pl.BlockSpec((block_n, d), lambda i: (i, 0)),
            pl.BlockSpec((p, d), lambda i: (0, 0)),
        ],
        out_specs=pl.BlockSpec((block_n, d), lambda i: (i, 0)),
        out_shape=jax.ShapeDtypeStruct((n, d), jnp.float32),
        scratch_shapes=[
            pltpu.VMEM((p, d), jnp.bfloat16),
            pltpu.VMEM((p, d), jnp.bfloat16),
        ],
    )(x, centroid_emb)


def kernel(x, centroid_emb):
    return _centroid_layer(x, centroid_emb)
